# baseline (speedup 1.0000x reference)
import functools

import jax
import jax.numpy as jnp
from jax import lax
from jax.experimental import pallas as pl
from jax.experimental.pallas import tpu as pltpu
from jax.experimental.pallas import tpu_sc as plsc

_D = 128
_K = 3
_BG = 1024


_BIG = 65536.0


def _topk_body(*refs, col_base, carry_in, emit_state):
    ns = 8 if carry_in else 0
    g_ref, f_ref, n2col_ref, n2row_ref = refs[:4]
    sin = refs[4:4 + ns]
    idx_ref = refs[4 + ns]
    sout = refs[5 + ns:]
    F = f_ref[...]
    NF = F.shape[0]
    G = g_ref[...] * jnp.bfloat16(-2.0)
    mm = lax.dot_general(G, F, (((1,), (1,)), ((), ())),
                         preferred_element_type=jnp.float32)
    g2 = n2col_ref[...]
    f2row = n2row_ref[...]
    NC = NF // _D
    inf = jnp.float32(jnp.inf)
    big = jnp.float32(_BIG)
    if carry_in:
        m1, m2, m3, zm = (sin[0][...], sin[1][...], sin[2][...], sin[3][...])
        c1, c2, c3, zc = (sin[4][...], sin[5][...], sin[6][...], sin[7][...])
    else:
        m1 = jnp.full((_BG, _D), inf)
        m2 = jnp.full((_BG, _D), inf)
        m3 = jnp.full((_BG, _D), inf)
        zm = jnp.full((_BG, _D), inf)
        c1 = jnp.full((_BG, _D), big)
        c2 = jnp.full((_BG, _D), big)
        c3 = jnp.full((_BG, _D), big)
        zc = jnp.full((_BG, _D), big)
    for nc in range(0, NC, 2):
        sa = (g2 + mm[:, nc * _D:(nc + 1) * _D]) \
            + f2row[:, nc * _D:(nc + 1) * _D]
        sb = (g2 + mm[:, (nc + 1) * _D:(nc + 2) * _D]) \
            + f2row[:, (nc + 1) * _D:(nc + 2) * _D]
        cca = jnp.float32(col_base + nc)
        ccb = jnp.float32(col_base + nc + 1)
        bit = sb < sa
        w = jnp.minimum(sa, sb)
        z = jnp.maximum(sa, sb)
        iw = jnp.where(bit, ccb, cca)
        iz = jnp.where(bit, cca, ccb)
        l1 = w < m1
        l2 = w < m2
        l3 = w < m3
        m3 = jnp.where(l3, jnp.where(l2, m2, w), m3)
        c3 = jnp.where(l3, jnp.where(l2, c2, iw), c3)
        m2 = jnp.where(l2, jnp.where(l1, m1, w), m2)
        c2 = jnp.where(l2, jnp.where(l1, c1, iw), c2)
        m1 = jnp.where(l1, w, m1)
        c1 = jnp.where(l1, iw, c1)
        lb = z < zm
        zm = jnp.where(lb, z, zm)
        zc = jnp.where(lb, iz, zc)

    if emit_state:
        for ref, arr in zip(sout, (m1, m2, m3, zm, c1, c2, c3, zc)):
            ref[...] = arr

    lanef = lax.broadcasted_iota(jnp.int32, (_BG, _D), 1).astype(jnp.float32)
    m = jnp.concatenate([m1, m2, m3, zm], axis=1)
    jf = jnp.concatenate([c1 * _D + lanef, c2 * _D + lanef,
                          c3 * _D + lanef, zc * _D + lanef], axis=1)
    idxs = []
    for r in range(_K):
        mv = jnp.min(m, axis=1, keepdims=True)
        cand = jnp.where(m == mv, jf, big)
        jwin = jnp.min(cand, axis=1, keepdims=True)
        idxs.append(jwin.astype(jnp.int32))
        if r < _K - 1:
            m = jnp.where(cand == jwin, inf, m)

    lane = lax.broadcasted_iota(jnp.int32, (_BG, _D), 1)
    idx_ref[...] = jnp.where(lane < 1, idxs[0],
                             jnp.where(lane < 2, idxs[1], idxs[2]))


def _topk_call(g16, f16, g2, f2, state=None, emit_state=False, col_base=0):
    NG, NF = g16.shape[0], f16.shape[0]
    grid = (NG // _BG,)
    gspec = pl.BlockSpec((_BG, _D), lambda i: (i, 0))
    in_specs = [
        gspec,
        pl.BlockSpec((NF, _D), lambda i: (0, 0)),
        pl.BlockSpec((_BG, 1), lambda i: (i, 0)),
        pl.BlockSpec((1, NF), lambda i: (0, 0)),
    ]
    args = [g16, f16, g2[:, None], f2[None, :]]
    if state is not None:
        in_specs += [gspec] * 8
        args += list(state)
    out_specs = [gspec]
    out_shape = [jax.ShapeDtypeStruct((NG, _D), jnp.int32)]
    if emit_state:
        out_specs += [gspec] * 8
        out_shape += [jax.ShapeDtypeStruct((NG, _D), jnp.float32)] * 8
    body = functools.partial(_topk_body, col_base=col_base,
                             carry_in=state is not None,
                             emit_state=emit_state)
    out = pl.pallas_call(
        body,
        grid=grid,
        in_specs=in_specs,
        out_specs=out_specs,
        out_shape=out_shape,
        compiler_params=pltpu.CompilerParams(
            dimension_semantics=("parallel",)),
    )(*args)
    if emit_state:
        return out[0], tuple(out[1:])
    return out[0]


def _sc_unpool(feats, i0, i1, i2):
    G = i0.shape[0]
    info = plsc.get_sparse_core_info()
    nw = info.num_cores * info.num_subcores
    C = G // nw
    mesh = plsc.VectorSubcoreMesh(core_axis_name="c", subcore_axis_name="s")

    @functools.partial(
        pl.kernel, mesh=mesh,
        out_type=jax.ShapeDtypeStruct((2 * G, _D), jnp.float32),
        scratch_types=[
            pltpu.VMEM((C,), jnp.int32),
            pltpu.VMEM((C,), jnp.int32),
            pltpu.VMEM((C,), jnp.int32),
            pltpu.VMEM((C, _D), jnp.float32),
            pltpu.VMEM((C, _D), jnp.float32),
            pltpu.VMEM((C, _D), jnp.float32),
            pltpu.SemaphoreType.DMA,
            pltpu.SemaphoreType.DMA,
        ],
    )
    def run(f_hbm, i0_hbm, i1_hbm, i2_hbm, out_hbm,
            i0v, i1v, i2v, r0, r1, r2, sem, semc):
        wid = lax.axis_index("s") * info.num_cores + lax.axis_index("c")
        base = wid * C
        cold = pltpu.async_copy(f_hbm.at[pl.ds(base, C)],
                                out_hbm.at[pl.ds(base, C)], semc)
        a0 = pltpu.async_copy(i0_hbm.at[pl.ds(base, C)], i0v, sem)
        a1 = pltpu.async_copy(i1_hbm.at[pl.ds(base, C)], i1v, sem)
        a2 = pltpu.async_copy(i2_hbm.at[pl.ds(base, C)], i2v, sem)
        a0.wait()
        a1.wait()
        a2.wait()
        c0 = pltpu.async_copy(f_hbm.at[i0v], r0, sem)
        c1 = pltpu.async_copy(f_hbm.at[i1v], r1, sem)
        c2 = pltpu.async_copy(f_hbm.at[i2v], r2, sem)
        c0.wait()
        c1.wait()
        c2.wait()

        def body(g, carry):
            for j in range(_D // 16):
                s = pl.ds(j * 16, 16)
                r0[g, s] = (r0[g, s] + r1[g, s] + r2[g, s]) / 3.0
            return carry

        lax.fori_loop(0, C, body, 0)
        pltpu.sync_copy(r0, out_hbm.at[pl.ds(G + base, C)])
        cold.wait()

    return run(feats, i0, i1, i2)


def kernel(x):
    N = x.shape[0]
    x16 = x.astype(jnp.bfloat16)
    n2a = jnp.sum(x * x, axis=1)

    idx1, st = _topk_call(x16, x16, n2a, n2a, emit_state=True)
    feats1 = _sc_unpool(x, idx1[:, 0], idx1[:, 1], idx1[:, 2])

    f16 = feats1.astype(jnp.bfloat16)
    n2 = jnp.sum(feats1 * feats1, axis=1)
    idx2a = _topk_call(f16[:N], f16[N:], n2[:N], n2[N:],
                       state=st, col_base=N // _D)
    idx2b = _topk_call(f16[N:], f16, n2[N:], n2)
    idx2 = jnp.concatenate([idx2a, idx2b], axis=0)
    return _sc_unpool(feats1, idx2[:, 0], idx2[:, 1], idx2[:, 2])


# speedup vs baseline: 1.5771x; 1.5771x over previous
import functools

import jax
import jax.numpy as jnp
from jax import lax
from jax.experimental import pallas as pl
from jax.experimental.pallas import tpu as pltpu
from jax.experimental.pallas import tpu_sc as plsc

_D = 128
_K = 3
_BG = 1024


_BIG = 65536.0


def _fresh_state():
    inf = jnp.float32(jnp.inf)
    big = jnp.float32(_BIG)
    return (jnp.full((_BG, _D), inf), jnp.full((_BG, _D), inf),
            jnp.full((_BG, _D), inf), jnp.full((_BG, _D), inf),
            jnp.full((_BG, _D), big), jnp.full((_BG, _D), big),
            jnp.full((_BG, _D), big), jnp.full((_BG, _D), big))


def _topk_body(g_ref, f_ref, g2_ref, f2_ref, *rest,
               col_base, carry_in, emit_state):
    ns = 8 if carry_in else 0
    srefs = rest[:ns]
    outs = rest[ns:]
    idx_ref = outs[0]
    sout = outs[1:]

    inf = jnp.float32(jnp.inf)
    big = jnp.float32(_BIG)

    gneg = g_ref[...] * jnp.bfloat16(-2)
    mm = lax.dot_general(gneg, f_ref[...], (((1,), (1,)), ((), ())),
                         preferred_element_type=jnp.float32)
    g2 = g2_ref[...]
    f2row = f2_ref[...]

    if carry_in:
        m1, m2, m3, zm, c1, c2, c3, zc = (r[...] for r in srefs)
    else:
        m1, m2, m3, zm, c1, c2, c3, zc = _fresh_state()

    NC = mm.shape[1] // _D
    for nc in range(0, NC, 2):
        sa = (g2 + mm[:, nc * _D:(nc + 1) * _D]) \
            + f2row[:, nc * _D:(nc + 1) * _D]
        sb = (g2 + mm[:, (nc + 1) * _D:(nc + 2) * _D]) \
            + f2row[:, (nc + 1) * _D:(nc + 2) * _D]
        cca = jnp.float32(col_base + nc)
        ccb = jnp.float32(col_base + nc + 1)
        bit = sb < sa
        w = jnp.minimum(sa, sb)
        z = jnp.maximum(sa, sb)
        iw = jnp.where(bit, ccb, cca)
        iz = jnp.where(bit, cca, ccb)
        l1 = w < m1
        l2 = w < m2
        l3 = w < m3
        m3 = jnp.where(l3, jnp.where(l2, m2, w), m3)
        c3 = jnp.where(l3, jnp.where(l2, c2, iw), c3)
        m2 = jnp.where(l2, jnp.where(l1, m1, w), m2)
        c2 = jnp.where(l2, jnp.where(l1, c1, iw), c2)
        m1 = jnp.where(l1, w, m1)
        c1 = jnp.where(l1, iw, c1)
        lb = z < zm
        zm = jnp.where(lb, z, zm)
        zc = jnp.where(lb, iz, zc)

    if emit_state:
        for ref, arr in zip(sout, (m1, m2, m3, zm, c1, c2, c3, zc)):
            ref[...] = arr

    lanef = lax.broadcasted_iota(jnp.int32, (_BG, _D), 1).astype(jnp.float32)
    m = jnp.concatenate([m1, m2, m3, zm], axis=1)
    jf = jnp.concatenate([c1 * _D + lanef, c2 * _D + lanef,
                          c3 * _D + lanef, zc * _D + lanef], axis=1)
    idxs = []
    for r in range(_K):
        mv = jnp.min(m, axis=1, keepdims=True)
        cand = jnp.where(m == mv, jf, big)
        jwin = jnp.min(cand, axis=1, keepdims=True)
        idxs.append(jwin.astype(jnp.int32))
        if r < _K - 1:
            m = jnp.where(cand == jwin, inf, m)

    lane = lax.broadcasted_iota(jnp.int32, (_BG, _D), 1)
    idx_ref[...] = jnp.where(lane < 1, idxs[0],
                             jnp.where(lane < 2, idxs[1], idxs[2]))


def _topk_call(g16, f16, g2, f2, state=None, emit_state=False, col_base=0):
    NG, NF = g16.shape[0], f16.shape[0]
    grid = (NG // _BG,)
    gspec = pl.BlockSpec((_BG, _D), lambda i: (i, 0))
    in_specs = [
        gspec,
        pl.BlockSpec((NF, _D), lambda i: (0, 0)),
        pl.BlockSpec((_BG, 1), lambda i: (i, 0)),
        pl.BlockSpec((1, NF), lambda i: (0, 0)),
    ]
    args = [g16, f16, g2[:, None], f2[None, :]]
    if state is not None:
        in_specs += [gspec] * 8
        args += list(state)
    out_specs = [gspec]
    out_shape = [jax.ShapeDtypeStruct((NG, _D), jnp.int32)]
    if emit_state:
        out_specs += [gspec] * 8
        out_shape += [jax.ShapeDtypeStruct((NG, _D), jnp.float32)] * 8
    body = functools.partial(_topk_body, col_base=col_base,
                             carry_in=state is not None,
                             emit_state=emit_state)
    out = pl.pallas_call(
        body,
        grid=grid,
        in_specs=in_specs,
        out_specs=out_specs,
        out_shape=out_shape,
        compiler_params=pltpu.CompilerParams(
            dimension_semantics=("parallel",)),
    )(*args)
    if emit_state:
        return out[0], tuple(out[1:])
    return out[0]


def _sc_gather_mean(feats, i0, i1, i2):
    G = i0.shape[0]
    info = plsc.get_sparse_core_info()
    nw = info.num_cores * info.num_subcores
    C = G // nw
    mesh = plsc.VectorSubcoreMesh(core_axis_name="c", subcore_axis_name="s")

    @functools.partial(
        pl.kernel, mesh=mesh,
        out_type=jax.ShapeDtypeStruct((G, _D), jnp.float32),
        scratch_types=[
            pltpu.VMEM((C,), jnp.int32),
            pltpu.VMEM((C,), jnp.int32),
            pltpu.VMEM((C,), jnp.int32),
            pltpu.VMEM((C, _D), jnp.float32),
            pltpu.VMEM((C, _D), jnp.float32),
            pltpu.VMEM((C, _D), jnp.float32),
            pltpu.SemaphoreType.DMA,
        ],
    )
    def run(f_hbm, i0_hbm, i1_hbm, i2_hbm, out_hbm,
            i0v, i1v, i2v, r0, r1, r2, sem):
        wid = lax.axis_index("s") * info.num_cores + lax.axis_index("c")
        base = wid * C
        pltpu.sync_copy(i0_hbm.at[pl.ds(base, C)], i0v)
        pltpu.sync_copy(i1_hbm.at[pl.ds(base, C)], i1v)
        pltpu.sync_copy(i2_hbm.at[pl.ds(base, C)], i2v)
        c0 = pltpu.async_copy(f_hbm.at[i0v], r0, sem)
        c1 = pltpu.async_copy(f_hbm.at[i1v], r1, sem)
        c2 = pltpu.async_copy(f_hbm.at[i2v], r2, sem)
        c0.wait()
        c1.wait()
        c2.wait()

        def body(g, carry):
            for j in range(_D // 16):
                s = pl.ds(j * 16, 16)
                r0[g, s] = (r0[g, s] + r1[g, s] + r2[g, s]) / 3.0
            return carry

        lax.fori_loop(0, C, body, 0)
        pltpu.sync_copy(r0, out_hbm.at[pl.ds(base, C)])

    return run(feats, i0, i1, i2)


def kernel(x):
    N = x.shape[0]
    x16 = x.astype(jnp.bfloat16)
    n2a = jnp.sum(x * x, axis=1)

    idx1, st = _topk_call(x16, x16, n2a, n2a, emit_state=True)
    new1 = _sc_gather_mean(x, idx1[:, 0], idx1[:, 1], idx1[:, 2])
    feats1 = jnp.concatenate([x, new1], axis=0)

    f16 = feats1.astype(jnp.bfloat16)
    n2 = jnp.sum(feats1 * feats1, axis=1)
    idx2a = _topk_call(f16[:N], f16[N:], n2[:N], n2[N:],
                       state=st, col_base=N // _D)
    idx2b = _topk_call(f16[N:], f16, n2[N:], n2)
    idx2 = jnp.concatenate([idx2a, idx2b], axis=0)
    new2 = _sc_gather_mean(feats1, idx2[:, 0], idx2[:, 1], idx2[:, 2])
    return jnp.concatenate([feats1, new2], axis=0)
